# baseline (speedup 1.0000x reference)
import functools

import jax
import jax.numpy as jnp
from jax import lax
from jax.experimental import pallas as pl
from jax.experimental.pallas import tpu as pltpu
from jax.experimental.pallas import tpu_sc as plsc

VOCAB = 1000000
EMBED = 64
HIDDEN = 64
CLASSES = 2
BATCH = 16384

_info = plsc.get_sparse_core_info()
_NC, _NS, _L = _info.num_cores, _info.num_subcores, _info.num_lanes
_NW = _NC * _NS
_B_PER_W = BATCH // _NW
_N_CHUNK = _B_PER_W // _L

_VBLK = 1024
_NBLK = 1000448 // _VBLK
_GROWS = 1000448 // 128
_GFLAT = _GROWS * 256


def _tc_logit_table(params, table_t):
    def body(par_ref, e_ref, o_ref):
        e = e_ref[...]
        m0 = par_ref[0, :]
        m1 = par_ref[1, :]
        g0 = jnp.dot(m0, e, precision=lax.Precision.HIGHEST) + par_ref[2, 0]
        g1 = jnp.dot(m1, e, precision=lax.Precision.HIGHEST) + par_ref[2, 1]
        o_ref[...] = jnp.concatenate(
            [g0.reshape(_VBLK // 128, 128), g1.reshape(_VBLK // 128, 128)],
            axis=1,
        )

    return pl.pallas_call(
        body,
        grid=(_NBLK,),
        in_specs=[
            pl.BlockSpec((3, EMBED), lambda j: (0, 0)),
            pl.BlockSpec((EMBED, _VBLK), lambda j: (0, j)),
        ],
        out_specs=pl.BlockSpec((_VBLK // 128, 256), lambda j: (j, 0)),
        out_shape=jax.ShapeDtypeStruct((_GROWS, 256), jnp.float32),
    )(params, table_t)


def _sc_gather_logits(gflat, idx2):
    mesh = plsc.VectorSubcoreMesh(core_axis_name="c", subcore_axis_name="s")

    @functools.partial(
        pl.kernel,
        mesh=mesh,
        compiler_params=pltpu.CompilerParams(use_tc_tiling_on_sc=False),
        out_type=jax.ShapeDtypeStruct((CLASSES, BATCH), jnp.float32),
        scratch_types=[
            pltpu.VMEM((_B_PER_W,), jnp.int32),
            pltpu.VMEM((_B_PER_W,), jnp.int32),
            pltpu.VMEM((_B_PER_W,), jnp.int32),
            pltpu.VMEM((_B_PER_W,), jnp.float32),
            pltpu.VMEM((_B_PER_W,), jnp.float32),
            pltpu.SemaphoreType.DMA,
        ],
    )
    def k(g_hbm, idx_hbm, out_hbm, idx_v, f0_v, f1_v, d0_v, d1_v, sem):
        wid = lax.axis_index("s") * _NC + lax.axis_index("c")
        base = wid * _B_PER_W
        pltpu.sync_copy(idx_hbm.at[wid], idx_v)

        def flatten(ch, carry):
            off = ch * _L
            v = idx_v[pl.ds(off, _L)]
            flat0 = ((v >> 7) << 8) + (v & 127)
            f0_v[pl.ds(off, _L)] = flat0
            f1_v[pl.ds(off, _L)] = flat0 + 128
            return carry

        lax.fori_loop(0, _N_CHUNK, flatten, 0)

        copies = []
        for q in range(_B_PER_W // 128):
            copies.append(pltpu.async_copy(
                g_hbm.at[f0_v.at[pl.ds(q * 128, 128)]],
                d0_v.at[pl.ds(q * 128, 128)], sem))
            copies.append(pltpu.async_copy(
                g_hbm.at[f1_v.at[pl.ds(q * 128, 128)]],
                d1_v.at[pl.ds(q * 128, 128)], sem))
        for cp in copies:
            cp.wait()

        pltpu.sync_copy(d0_v, out_hbm.at[0, pl.ds(base, _B_PER_W)])
        pltpu.sync_copy(d1_v, out_hbm.at[1, pl.ds(base, _B_PER_W)])

    return k(gflat, idx2)


def kernel(word_ids, embedding, W1, b1, W2, b2):
    M = jnp.dot(W1, W2, precision=lax.Precision.HIGHEST)
    c = jnp.dot(b1, W2, precision=lax.Precision.HIGHEST) + b2
    params = jnp.zeros((3, EMBED), jnp.float32)
    params = params.at[0].set(M[:, 0])
    params = params.at[1].set(M[:, 1])
    params = params.at[2, 0].set(c[0])
    params = params.at[2, 1].set(c[1])
    g = _tc_logit_table(params, embedding.T)
    gflat = g.reshape(_GFLAT)
    idx2 = word_ids.astype(jnp.int32).reshape(_NW, _B_PER_W)
    out_t = _sc_gather_logits(gflat, idx2)
    return out_t.T
